# baseline (speedup 1.0000x reference)
import functools

import jax
import jax.numpy as jnp
from jax import lax
from jax.experimental import pallas as pl
from jax.experimental.pallas import tpu as pltpu
from jax.experimental.pallas import tpu_sc as plsc

NUM_EMB = 1024
IN_DIM = 64
BETA = 0.25
BB = 8


def _idx_kernel(x_ref, emb_ref, idx_ref, loss_ref):
    b = pl.program_id(0)
    emb = emb_ref[...]

    @pl.when(b == 0)
    def _init():
        loss_ref[...] = jnp.zeros_like(loss_ref)

    b2 = jnp.sum(emb * emb, axis=1, keepdims=True)
    emb2 = emb + emb
    iota_col = jax.lax.broadcasted_iota(
        jnp.int32, (NUM_EMB, 1), 0).astype(jnp.float32)

    acc = jnp.zeros((1, 1), jnp.float32)
    for i in range(BB):
        x = x_ref[i]
        a2 = jnp.sum(x * x, axis=0, keepdims=True)
        m2 = jax.lax.dot_general(
            emb2, x, (((1,), (0,)), ((), ())),
            preferred_element_type=jnp.float32)
        d2 = (a2 + b2) - m2

        dmin = jnp.min(d2, axis=0, keepdims=True)
        idx_f = jnp.min(jnp.where(d2 == dmin, iota_col, float(NUM_EMB)),
                        axis=0)
        idx_ref[i] = idx_f.astype(jnp.int32)
        acc = acc + jnp.sum(jnp.maximum(dmin, 0.0), keepdims=True).reshape(1, 1)

    loss_ref[...] += acc

    @pl.when(b == pl.num_programs(0) - 1)
    def _finalize():
        mean_sq = loss_ref[...] * (1.0 / (16 * IN_DIM * 1024))
        loss_ref[...] = mean_sq + BETA * mean_sq


def _xpose_kernel(xqf_ref, xq_ref):
    xq_ref[0] = xqf_ref[...][:, :IN_DIM].T


_SC_INFO = plsc.get_sparse_core_info()
_NC = _SC_INFO.num_cores
_NW = _NC * _SC_INFO.num_subcores


def _gather_kernel(b_per_w, table_hbm, idx_hbm, out_hbm, idx_v, rows_v, sem):
    wid = lax.axis_index("s") * _NC + lax.axis_index("c")
    base = wid * b_per_w
    pltpu.sync_copy(idx_hbm.at[pl.ds(base, b_per_w)], idx_v)
    pltpu.async_copy(table_hbm.at[idx_v], rows_v, sem).wait()
    pltpu.sync_copy(rows_v, out_hbm.at[pl.ds(base, b_per_w)])


@jax.jit
def kernel(x_in, emb):
    B, C, L = x_in.shape
    idxs, vq_loss2 = pl.pallas_call(
        _idx_kernel,
        grid=(B // BB,),
        in_specs=[
            pl.BlockSpec((BB, C, L), lambda b: (b, 0, 0)),
            pl.BlockSpec((NUM_EMB, IN_DIM), lambda b: (0, 0)),
        ],
        out_specs=[
            pl.BlockSpec((BB, L), lambda b: (b, 0)),
            pl.BlockSpec((1, 1), lambda b: (0, 0)),
        ],
        out_shape=[
            jax.ShapeDtypeStruct((B, L), jnp.int32),
            jax.ShapeDtypeStruct((1, 1), jnp.float32),
        ],
    )(x_in, emb)

    n_tok = B * L
    b_per_w = n_tok // _NW
    mesh = plsc.VectorSubcoreMesh(core_axis_name="c", subcore_axis_name="s")
    x_q_flat = pl.kernel(
        functools.partial(_gather_kernel, b_per_w),
        mesh=mesh,
        out_type=jax.ShapeDtypeStruct((n_tok, 128), jnp.float32),
        scratch_types=[
            pltpu.VMEM((b_per_w,), jnp.int32),
            pltpu.VMEM((b_per_w, 128), jnp.float32),
            pltpu.SemaphoreType.DMA,
        ],
    )(jnp.pad(emb, ((0, 0), (0, 128 - C))), idxs.reshape(n_tok))

    x_q = pl.pallas_call(
        _xpose_kernel,
        grid=(B,),
        in_specs=[pl.BlockSpec((L, 128), lambda b: (b, 0))],
        out_specs=pl.BlockSpec((1, C, L), lambda b: (b, 0, 0)),
        out_shape=jax.ShapeDtypeStruct((B, C, L), jnp.float32),
    )(x_q_flat)

    return (x_q, idxs, vq_loss2[0, 0])
